# Initial kernel scaffold; baseline (speedup 1.0000x reference)
import jax
import jax.numpy as jnp
from jax.experimental import pallas as pl


def kernel(input_ids, word_emb, pos_emb, ln_gamma, ln_beta):
    raise NotImplementedError("write your pallas kernel here")


# speedup vs baseline: 3.6555x; 3.6555x over previous
import functools

import jax
import jax.numpy as jnp
from jax import lax
from jax.experimental import pallas as pl
from jax.experimental.pallas import tpu as pltpu
from jax.experimental.pallas import tpu_sc as plsc

_EPS = 1e-12
_NW = 32
_CHUNK = 128


def _sc_gather(word_emb, ids_flat):
    n = ids_flat.shape[0]
    hid = word_emb.shape[1]
    per_w = n // _NW
    steps = per_w // _CHUNK
    mesh = plsc.VectorSubcoreMesh(core_axis_name="c", subcore_axis_name="s")

    @functools.partial(
        pl.kernel,
        mesh=mesh,
        out_type=jax.ShapeDtypeStruct((n, hid), jnp.float32),
        scratch_types=[
            pltpu.VMEM((per_w,), jnp.int32),
            pltpu.VMEM((_CHUNK, hid), jnp.float32),
            pltpu.VMEM((_CHUNK, hid), jnp.float32),
            pltpu.SemaphoreType.DMA,
            pltpu.SemaphoreType.DMA,
        ],
    )
    def k(tab_hbm, idx_hbm, out_hbm, idx_v, buf0, buf1, gsem, osem):
        wid = lax.axis_index("s") * 2 + lax.axis_index("c")
        base = pl.multiple_of(wid * per_w, 8)
        pltpu.sync_copy(idx_hbm.at[pl.ds(base, per_w)], idx_v)

        @pl.loop(0, steps, step=2)
        def _(j):
            o0 = pl.multiple_of(base + j * _CHUNK, 8)
            o1 = pl.multiple_of(base + (j + 1) * _CHUNK, 8)
            pltpu.async_copy(
                tab_hbm.at[idx_v.at[pl.ds(j * _CHUNK, _CHUNK)]], buf0,
                gsem).wait()
            w0 = pltpu.async_copy(buf0, out_hbm.at[pl.ds(o0, _CHUNK)], osem)
            pltpu.async_copy(
                tab_hbm.at[idx_v.at[pl.ds((j + 1) * _CHUNK, _CHUNK)]], buf1,
                gsem).wait()
            w0.wait()
            pltpu.async_copy(buf1, out_hbm.at[pl.ds(o1, _CHUNK)], osem).wait()

    return k(word_emb, ids_flat)


def _tc_ln(gathered, pos_tiled, gamma, beta, rows, hid, rblk):
    grid = (rows // rblk,)

    def body(x_ref, pos_ref, g_ref, b_ref, o_ref):
        x = x_ref[...] + pos_ref[...]
        mean = jnp.mean(x, axis=-1, keepdims=True)
        var = jnp.mean((x - mean) ** 2, axis=-1, keepdims=True)
        y = (x - mean) * lax.rsqrt(var + _EPS)
        o_ref[...] = y * g_ref[...][0] + b_ref[...][0]

    return pl.pallas_call(
        body,
        grid=grid,
        in_specs=[
            pl.BlockSpec((rblk, hid), lambda i: (i, 0)),
            pl.BlockSpec((rblk, hid), lambda i: (0, 0)),
            pl.BlockSpec((1, hid), lambda i: (0, 0)),
            pl.BlockSpec((1, hid), lambda i: (0, 0)),
        ],
        out_specs=pl.BlockSpec((rblk, hid), lambda i: (i, 0)),
        out_shape=jax.ShapeDtypeStruct((rows, hid), jnp.float32),
    )(gathered, pos_tiled, gamma.reshape(1, hid), beta.reshape(1, hid))


def kernel(input_ids, word_emb, pos_emb, ln_gamma, ln_beta):
    b, s = input_ids.shape
    hid = word_emb.shape[1]
    rows = b * s
    gathered = _sc_gather(word_emb, input_ids.reshape(-1))
    rblk = 8 * s
    pos_tiled = jnp.tile(pos_emb[:s], (8, 1))
    out = _tc_ln(gathered, pos_tiled, ln_gamma, ln_beta, rows, hid, rblk)
    return out.reshape(b, s, hid)
